# baseline (speedup 1.0000x reference)
import jax
import jax.numpy as jnp
from jax.experimental import pallas as pl

EPS = 1e-12
BLK = 1024


def _ln_kernel(x_ref, pos_ref, gamma_ref, beta_ref, out_ref):
    h = x_ref.shape[-1]
    pos = pos_ref[...]
    gamma = gamma_ref[...]
    beta = beta_ref[...]
    ones = jnp.ones((h, 128), dtype=jnp.bfloat16)
    for bi in range(x_ref.shape[0]):
        e = x_ref[bi] + pos
        eb = e.astype(jnp.bfloat16)
        s1 = jax.lax.dot_general(
            eb, ones, (((1,), (0,)), ((), ())),
            preferred_element_type=jnp.float32)[:, :1]
        s2 = jax.lax.dot_general(
            eb * eb, ones, (((1,), (0,)), ((), ())),
            preferred_element_type=jnp.float32)[:, :1]
        mean = s1 * (1.0 / h)
        var = s2 * (1.0 / h) - mean * mean
        inv = jax.lax.rsqrt(var + EPS)
        out_ref[bi] = (e - mean) * inv * gamma + beta


def kernel(x, pos_table, gamma, beta):
    b, s, hdim = x.shape
    gamma2 = gamma.reshape(1, hdim)
    beta2 = beta.reshape(1, hdim)
    grid = (s // BLK,)
    return pl.pallas_call(
        _ln_kernel,
        grid=grid,
        in_specs=[
            pl.BlockSpec((b, BLK, hdim), lambda i: (0, i, 0)),
            pl.BlockSpec((BLK, hdim), lambda i: (i, 0)),
            pl.BlockSpec((1, hdim), lambda i: (0, 0)),
            pl.BlockSpec((1, hdim), lambda i: (0, 0)),
        ],
        out_specs=pl.BlockSpec((b, BLK, hdim), lambda i: (0, i, 0)),
        out_shape=jax.ShapeDtypeStruct((b, s, hdim), x.dtype),
    )(x, pos_table, gamma2, beta2)
